# Initial kernel scaffold; baseline (speedup 1.0000x reference)
import functools

import jax
import jax.numpy as jnp
from jax import lax
from jax.experimental import pallas as pl
from jax.experimental.pallas import tpu as pltpu
from jax.experimental.pallas import tpu_sc as plsc

N_NODES = 10000
N_EDGES = 320000
D = 128

NC = 2
NS = 16
NW = NC * NS
L = 16

C = 128
N_PAD = 10240
E_PAD = 327680
ROWS_PER_TILE = N_PAD // NS
ECH_G = E_PAD // NW // C
ECH_SC = E_PAD // NS // C

_mesh = plsc.VectorSubcoreMesh(core_axis_name="c", subcore_axis_name="s")


def _rsqrt_newton(x):
    i = plsc.bitcast(x, jnp.int32)
    y = plsc.bitcast(jnp.int32(0x5F3759DF) - (i >> 1), jnp.float32)
    for _ in range(3):
        y = y * (1.5 - 0.5 * x * y * y)
    return y


@functools.partial(
    pl.kernel,
    out_type=(
        jax.ShapeDtypeStruct((E_PAD // C, C), jnp.float32),
        jax.ShapeDtypeStruct((N_PAD,), jnp.float32),
    ),
    mesh=_mesh,
    scratch_types=(
        pltpu.VMEM((ECH_SC, C), jnp.int32),
        pltpu.VMEM((ECH_SC, C), jnp.float32),
        pltpu.VMEM((ECH_G, C), jnp.int32),
        pltpu.VMEM((ECH_G, C), jnp.float32),
        pltpu.VMEM((N_PAD,), jnp.float32),
        pltpu.VMEM((N_PAD // 16,), jnp.float32),
        pltpu.VMEM_SHARED((N_PAD,), jnp.float32),
        pltpu.VMEM_SHARED((N_PAD,), jnp.float32),
    ),
)
def _prep(row_hbm, col_hbm, ew_hbm, norm_hbm, sw_hbm,
          col_v, ew_v, row_v, norm_v, dis_v, node_v, deg_sp, dis_sp):
    cid = lax.axis_index("c")
    sid = lax.axis_index("s")
    wid = sid * NC + cid

    def _init_body(k, _):
        node_v[pl.ds(k * L, L)] = jnp.full((L,), 1.0, jnp.float32)
        return 0
    lax.fori_loop(0, ROWS_PER_TILE // L, _init_body, 0)
    pltpu.sync_copy(node_v, deg_sp.at[pl.ds(sid * ROWS_PER_TILE, ROWS_PER_TILE)])
    plsc.subcore_barrier()

    pltpu.sync_copy(col_hbm.at[pl.ds(sid * ECH_SC, ECH_SC)], col_v)
    pltpu.sync_copy(ew_hbm.at[pl.ds(sid * ECH_SC, ECH_SC)], ew_v)

    def _deg_body(j, _):
        pltpu.sync_copy(ew_v.at[j], deg_sp.at[col_v.at[j]], add=True)
        return 0
    lax.fori_loop(0, ECH_SC, _deg_body, 0)
    plsc.subcore_barrier()

    base = sid * ROWS_PER_TILE
    pltpu.sync_copy(deg_sp.at[pl.ds(base, ROWS_PER_TILE)], node_v)

    def _dis_body(k, _):
        x = node_v[pl.ds(k * L, L)]
        node_v[pl.ds(k * L, L)] = _rsqrt_newton(x)
        return 0
    lax.fori_loop(0, ROWS_PER_TILE // L, _dis_body, 0)
    pltpu.sync_copy(node_v, dis_sp.at[pl.ds(base, ROWS_PER_TILE)])

    @pl.when(cid == 0)
    def _():
        def _sw_body(k, _):
            y = node_v[pl.ds(k * L, L)]
            node_v[pl.ds(k * L, L)] = y * y
            return 0
        lax.fori_loop(0, ROWS_PER_TILE // L, _sw_body, 0)
        pltpu.sync_copy(node_v, sw_hbm.at[pl.ds(base, ROWS_PER_TILE)])
    plsc.subcore_barrier()

    pltpu.sync_copy(dis_sp, dis_v)
    ebase = wid * ECH_G
    pltpu.sync_copy(row_hbm.at[pl.ds(ebase, ECH_G)], row_v)
    pltpu.sync_copy(col_hbm.at[pl.ds(ebase, ECH_G)], col_v.at[pl.ds(0, ECH_G)])
    pltpu.sync_copy(ew_hbm.at[pl.ds(ebase, ECH_G)], ew_v.at[pl.ds(0, ECH_G)])

    def _norm_body(j, _):
        def _inner(m, _):
            r = row_v[j, pl.ds(m * L, L)]
            c = col_v[j, pl.ds(m * L, L)]
            w = ew_v[j, pl.ds(m * L, L)]
            dr = plsc.load_gather(dis_v, [r])
            dc = plsc.load_gather(dis_v, [c])
            norm_v[j, pl.ds(m * L, L)] = dr * w * dc
            return 0
        lax.fori_loop(0, C // L, _inner, 0)
        return 0
    lax.fori_loop(0, ECH_G, _norm_body, 0)
    pltpu.sync_copy(norm_v, norm_hbm.at[pl.ds(ebase, ECH_G)])


@functools.partial(
    pl.kernel,
    out_type=jax.ShapeDtypeStruct((NC, N_PAD, D), jnp.float32),
    mesh=_mesh,
    scratch_types=(
        pltpu.VMEM((ECH_G, C), jnp.int32),
        pltpu.VMEM((ECH_G, C), jnp.int32),
        pltpu.VMEM((ECH_G, C), jnp.float32),
        pltpu.VMEM((C, D), jnp.float32),
        pltpu.VMEM_SHARED((N_PAD, D), jnp.float32),
        pltpu.SemaphoreType.DMA,
    ),
)
def _hop(h_hbm, row_hbm, col_hbm, norm_hbm, out_hbm,
         row_v, col_v, norm_v, rows_v, acc_sp, sem):
    cid = lax.axis_index("c")
    sid = lax.axis_index("s")
    wid = sid * NC + cid

    def _z_body(i, _):
        for cc in range(D // L):
            rows_v[i, pl.ds(cc * L, L)] = jnp.zeros((L,), jnp.float32)
        return 0
    lax.fori_loop(0, C, _z_body, 0)
    nbase = sid * ROWS_PER_TILE
    for t in range(ROWS_PER_TILE // C):
        pltpu.sync_copy(rows_v, acc_sp.at[pl.ds(nbase + t * C, C)])
    plsc.subcore_barrier()

    ebase = wid * ECH_G
    pltpu.sync_copy(row_hbm.at[pl.ds(ebase, ECH_G)], row_v)
    pltpu.sync_copy(col_hbm.at[pl.ds(ebase, ECH_G)], col_v)
    pltpu.sync_copy(norm_hbm.at[pl.ds(ebase, ECH_G)], norm_v)

    def _chunk_body(j, _):
        pltpu.async_copy(h_hbm.at[row_v.at[j]], rows_v, sem).wait()

        def _scale_body(i, _):
            s = norm_v[j, i]
            for cc in range(D // L):
                rows_v[i, pl.ds(cc * L, L)] = rows_v[i, pl.ds(cc * L, L)] * s
            return 0
        lax.fori_loop(0, C, _scale_body, 0)
        pltpu.sync_copy(rows_v, acc_sp.at[col_v.at[j]], add=True)
        return 0
    lax.fori_loop(0, ECH_G, _chunk_body, 0)
    plsc.subcore_barrier()

    pltpu.sync_copy(acc_sp.at[pl.ds(nbase, ROWS_PER_TILE)],
                    out_hbm.at[cid].at[pl.ds(nbase, ROWS_PER_TILE)])


_ROWS_BLK = 1024


def _combine_body(p_ref, sw_ref, h_ref, o_ref):
    o_ref[...] = p_ref[0] + p_ref[1] + sw_ref[...] * h_ref[...]


def _final_body(p_ref, sw_ref, h_ref, w_ref, b_ref, o_ref):
    h2 = p_ref[0] + p_ref[1] + sw_ref[...] * h_ref[...]
    o_ref[...] = lax.dot_general(
        h2, w_ref[...], (((1,), (1,)), ((), ())),
        preferred_element_type=jnp.float32) + b_ref[...]


def _combine(p, sw, h):
    grid = (N_PAD // _ROWS_BLK,)
    return pl.pallas_call(
        _combine_body,
        grid=grid,
        in_specs=[
            pl.BlockSpec((NC, _ROWS_BLK, D), lambda i: (0, i, 0)),
            pl.BlockSpec((_ROWS_BLK, 1), lambda i: (i, 0)),
            pl.BlockSpec((_ROWS_BLK, D), lambda i: (i, 0)),
        ],
        out_specs=pl.BlockSpec((_ROWS_BLK, D), lambda i: (i, 0)),
        out_shape=jax.ShapeDtypeStruct((N_PAD, D), jnp.float32),
    )(p, sw, h)


def _final(p, sw, h, W, b):
    grid = (N_PAD // _ROWS_BLK,)
    return pl.pallas_call(
        _final_body,
        grid=grid,
        in_specs=[
            pl.BlockSpec((NC, _ROWS_BLK, D), lambda i: (0, i, 0)),
            pl.BlockSpec((_ROWS_BLK, 1), lambda i: (i, 0)),
            pl.BlockSpec((_ROWS_BLK, D), lambda i: (i, 0)),
            pl.BlockSpec((D, D), lambda i: (0, 0)),
            pl.BlockSpec((1, D), lambda i: (0, 0)),
        ],
        out_specs=pl.BlockSpec((_ROWS_BLK, D), lambda i: (i, 0)),
        out_shape=jax.ShapeDtypeStruct((N_PAD, D), jnp.float32),
    )(p, sw, h, W, b)


@jax.jit
def kernel(x, edge_index, edge_weight, W, b):
    npad = E_PAD - N_EDGES
    pad_idx = (jnp.arange(npad, dtype=jnp.int32) * 37) % N_NODES
    row = jnp.concatenate([edge_index[0].astype(jnp.int32), pad_idx])
    col = jnp.concatenate([edge_index[1].astype(jnp.int32), pad_idx])
    ew = jnp.concatenate([edge_weight, jnp.zeros((npad,), jnp.float32)])
    row2 = row.reshape(E_PAD // C, C)
    col2 = col.reshape(E_PAD // C, C)
    ew2 = ew.reshape(E_PAD // C, C)

    norm, sw = _prep(row2, col2, ew2)
    sw2 = sw.reshape(N_PAD, 1)

    h0 = jnp.zeros((N_PAD, D), jnp.float32).at[:N_NODES].set(x)
    p = _hop(h0, row2, col2, norm)
    h1 = _combine(p, sw2, h0)
    q = _hop(h1, row2, col2, norm)
    out = _final(q, sw2, h1, W, b)
    return out[:N_NODES]


# speedup vs baseline: 18.0647x; 18.0647x over previous
import functools

import jax
import jax.numpy as jnp
from jax import lax
from jax.experimental import pallas as pl
from jax.experimental.pallas import tpu as pltpu
from jax.experimental.pallas import tpu_sc as plsc

N_NODES = 10000
N_EDGES = 320000
D = 128

NC = 2
NS = 16
NW = NC * NS
L = 16

C = 128
N_PAD = 10240
E_PAD = 327680
ROWS_PER_TILE = N_PAD // NS
ECH_G = E_PAD // NW // C
ECH_SC = E_PAD // NS // C

_mesh = plsc.VectorSubcoreMesh(core_axis_name="c", subcore_axis_name="s")


def _rsqrt_newton(x):
    i = lax.bitcast_convert_type(x, jnp.int32)
    y = lax.bitcast_convert_type(jnp.int32(0x5F3759DF) - (i >> 1), jnp.float32)
    for _ in range(3):
        y = y * (1.5 - 0.5 * x * y * y)
    return y


@functools.partial(
    pl.kernel,
    out_type=(
        jax.ShapeDtypeStruct((E_PAD // C, C), jnp.float32),
        jax.ShapeDtypeStruct((N_PAD,), jnp.float32),
    ),
    mesh=_mesh,
    compiler_params=pltpu.CompilerParams(needs_layout_passes=False),
    scratch_types=(
        pltpu.VMEM((ECH_SC, C), jnp.int32),
        pltpu.VMEM((ECH_SC, C), jnp.float32),
        pltpu.VMEM((ECH_G, C), jnp.int32),
        pltpu.VMEM((ECH_G, C), jnp.float32),
        pltpu.VMEM((N_PAD,), jnp.float32),
        pltpu.VMEM((N_PAD // 16,), jnp.float32),
        pltpu.VMEM_SHARED((N_PAD,), jnp.float32),
        pltpu.VMEM_SHARED((N_PAD,), jnp.float32),
    ),
)
def _prep(row_hbm, col_hbm, ew_hbm, norm_hbm, sw_hbm,
          col_v, ew_v, row_v, norm_v, dis_v, node_v, deg_sp, dis_sp):
    cid = lax.axis_index("c")
    sid = lax.axis_index("s")
    wid = sid * NC + cid

    def _init_body(k, _):
        node_v[pl.ds(k * L, L)] = jnp.full((L,), 1.0, jnp.float32)
        return 0
    lax.fori_loop(0, ROWS_PER_TILE // L, _init_body, 0)
    pltpu.sync_copy(node_v, deg_sp.at[pl.ds(sid * ROWS_PER_TILE, ROWS_PER_TILE)])
    plsc.subcore_barrier()

    pltpu.sync_copy(col_hbm.at[pl.ds(sid * ECH_SC, ECH_SC)], col_v)
    pltpu.sync_copy(ew_hbm.at[pl.ds(sid * ECH_SC, ECH_SC)], ew_v)

    def _deg_body(j, _):
        pltpu.sync_copy(ew_v.at[j], deg_sp.at[col_v.at[j]], add=True)
        return 0
    lax.fori_loop(0, ECH_SC, _deg_body, 0)
    plsc.subcore_barrier()

    base = sid * ROWS_PER_TILE
    pltpu.sync_copy(deg_sp.at[pl.ds(base, ROWS_PER_TILE)], node_v)

    def _dis_body(k, _):
        x = node_v[pl.ds(k * L, L)]
        node_v[pl.ds(k * L, L)] = _rsqrt_newton(x)
        return 0
    lax.fori_loop(0, ROWS_PER_TILE // L, _dis_body, 0)
    pltpu.sync_copy(node_v, dis_sp.at[pl.ds(base, ROWS_PER_TILE)])

    @pl.when(cid == 0)
    def _():
        def _sw_body(k, _):
            y = node_v[pl.ds(k * L, L)]
            node_v[pl.ds(k * L, L)] = y * y
            return 0
        lax.fori_loop(0, ROWS_PER_TILE // L, _sw_body, 0)
        pltpu.sync_copy(node_v, sw_hbm.at[pl.ds(base, ROWS_PER_TILE)])
    plsc.subcore_barrier()

    pltpu.sync_copy(dis_sp, dis_v)
    ebase = wid * ECH_G
    pltpu.sync_copy(row_hbm.at[pl.ds(ebase, ECH_G)], row_v)
    pltpu.sync_copy(col_hbm.at[pl.ds(ebase, ECH_G)], col_v.at[pl.ds(0, ECH_G)])
    pltpu.sync_copy(ew_hbm.at[pl.ds(ebase, ECH_G)], ew_v.at[pl.ds(0, ECH_G)])

    def _norm_body(j, _):
        def _inner(m, _):
            r = row_v[j, pl.ds(m * L, L)]
            c = col_v[j, pl.ds(m * L, L)]
            w = ew_v[j, pl.ds(m * L, L)]
            dr = plsc.load_gather(dis_v, [r])
            dc = plsc.load_gather(dis_v, [c])
            norm_v[j, pl.ds(m * L, L)] = dr * w * dc
            return 0
        lax.fori_loop(0, C // L, _inner, 0)
        return 0
    lax.fori_loop(0, ECH_G, _norm_body, 0)
    pltpu.sync_copy(norm_v, norm_hbm.at[pl.ds(ebase, ECH_G)])


@functools.partial(
    pl.kernel,
    out_type=jax.ShapeDtypeStruct((NC, N_PAD, D), jnp.float32),
    mesh=_mesh,
    compiler_params=pltpu.CompilerParams(needs_layout_passes=False),
    scratch_types=(
        pltpu.VMEM((ECH_G, C), jnp.int32),
        pltpu.VMEM((ECH_G, C), jnp.int32),
        pltpu.VMEM((ECH_G, C), jnp.float32),
        pltpu.VMEM((C, D), jnp.float32),
        pltpu.VMEM_SHARED((N_PAD, D), jnp.float32),
        pltpu.SemaphoreType.DMA,
    ),
)
def _hop(h_hbm, row_hbm, col_hbm, norm_hbm, out_hbm,
         row_v, col_v, norm_v, rows_v, acc_sp, sem):
    cid = lax.axis_index("c")
    sid = lax.axis_index("s")
    wid = sid * NC + cid

    def _z_body(i, _):
        for cc in range(D // L):
            rows_v[i, pl.ds(cc * L, L)] = jnp.zeros((L,), jnp.float32)
        return 0
    lax.fori_loop(0, C, _z_body, 0)
    nbase = sid * ROWS_PER_TILE
    for t in range(ROWS_PER_TILE // C):
        pltpu.sync_copy(rows_v, acc_sp.at[pl.ds(nbase + t * C, C)])
    plsc.subcore_barrier()

    ebase = wid * ECH_G
    pltpu.sync_copy(row_hbm.at[pl.ds(ebase, ECH_G)], row_v)
    pltpu.sync_copy(col_hbm.at[pl.ds(ebase, ECH_G)], col_v)
    pltpu.sync_copy(norm_hbm.at[pl.ds(ebase, ECH_G)], norm_v)

    def _chunk_body(j, _):
        pltpu.async_copy(h_hbm.at[row_v.at[j]], rows_v, sem).wait()

        def _scale_body(g, _):
            nv = norm_v[j, pl.ds(g * L, L)]
            for k in range(L):
                s = nv[k]
                r = g * L + k
                for cc in range(D // L):
                    rows_v[r, pl.ds(cc * L, L)] = rows_v[r, pl.ds(cc * L, L)] * s
            return 0
        lax.fori_loop(0, C // L, _scale_body, 0)
        pltpu.sync_copy(rows_v, acc_sp.at[col_v.at[j]], add=True)
        return 0
    lax.fori_loop(0, ECH_G, _chunk_body, 0)
    plsc.subcore_barrier()

    pltpu.sync_copy(acc_sp.at[pl.ds(nbase, ROWS_PER_TILE)],
                    out_hbm.at[cid].at[pl.ds(nbase, ROWS_PER_TILE)])


_ROWS_BLK = 1024


def _combine_body(p_ref, sw_ref, h_ref, o_ref):
    o_ref[...] = p_ref[0] + p_ref[1] + sw_ref[...] * h_ref[...]


def _final_body(p_ref, sw_ref, h_ref, w_ref, b_ref, o_ref):
    h2 = p_ref[0] + p_ref[1] + sw_ref[...] * h_ref[...]
    o_ref[...] = lax.dot_general(
        h2, w_ref[...], (((1,), (1,)), ((), ())),
        preferred_element_type=jnp.float32) + b_ref[...]


def _combine(p, sw, h):
    grid = (N_PAD // _ROWS_BLK,)
    return pl.pallas_call(
        _combine_body,
        grid=grid,
        in_specs=[
            pl.BlockSpec((NC, _ROWS_BLK, D), lambda i: (0, i, 0)),
            pl.BlockSpec((_ROWS_BLK, 1), lambda i: (i, 0)),
            pl.BlockSpec((_ROWS_BLK, D), lambda i: (i, 0)),
        ],
        out_specs=pl.BlockSpec((_ROWS_BLK, D), lambda i: (i, 0)),
        out_shape=jax.ShapeDtypeStruct((N_PAD, D), jnp.float32),
    )(p, sw, h)


def _final(p, sw, h, W, b):
    grid = (N_PAD // _ROWS_BLK,)
    return pl.pallas_call(
        _final_body,
        grid=grid,
        in_specs=[
            pl.BlockSpec((NC, _ROWS_BLK, D), lambda i: (0, i, 0)),
            pl.BlockSpec((_ROWS_BLK, 1), lambda i: (i, 0)),
            pl.BlockSpec((_ROWS_BLK, D), lambda i: (i, 0)),
            pl.BlockSpec((D, D), lambda i: (0, 0)),
            pl.BlockSpec((1, D), lambda i: (0, 0)),
        ],
        out_specs=pl.BlockSpec((_ROWS_BLK, D), lambda i: (i, 0)),
        out_shape=jax.ShapeDtypeStruct((N_PAD, D), jnp.float32),
    )(p, sw, h, W, b)


@jax.jit
def kernel(x, edge_index, edge_weight, W, b):
    npad = E_PAD - N_EDGES
    pad_idx = (jnp.arange(npad, dtype=jnp.int32) * 37) % N_NODES
    row = jnp.concatenate([edge_index[0].astype(jnp.int32), pad_idx])
    col = jnp.concatenate([edge_index[1].astype(jnp.int32), pad_idx])
    ew = jnp.concatenate([edge_weight, jnp.zeros((npad,), jnp.float32)])
    row2 = row.reshape(E_PAD // C, C)
    col2 = col.reshape(E_PAD // C, C)
    ew2 = ew.reshape(E_PAD // C, C)

    norm, sw = _prep(row2, col2, ew2)
    sw2 = sw.reshape(N_PAD, 1)

    h0 = jnp.zeros((N_PAD, D), jnp.float32).at[:N_NODES].set(x)
    p = _hop(h0, row2, col2, norm)
    h1 = _combine(p, sw2, h0)
    q = _hop(h1, row2, col2, norm)
    out = _final(q, sw2, h1, W, b.reshape(1, D))
    return out[:N_NODES]


# speedup vs baseline: 21.9268x; 1.2138x over previous
import functools

import jax
import jax.numpy as jnp
from jax import lax
from jax.experimental import pallas as pl
from jax.experimental.pallas import tpu as pltpu
from jax.experimental.pallas import tpu_sc as plsc

N_NODES = 10000
N_EDGES = 320000
D = 128

NC = 2
NS = 16
NW = NC * NS
L = 16

C = 128
N_PAD = 10240
E_PAD = 327680
ROWS_PER_TILE = N_PAD // NS
ECH_G = E_PAD // NW // C
ECH_SC = E_PAD // NS // C

_mesh = plsc.VectorSubcoreMesh(core_axis_name="c", subcore_axis_name="s")


def _rsqrt_newton(x):
    i = lax.bitcast_convert_type(x, jnp.int32)
    y = lax.bitcast_convert_type(jnp.int32(0x5F3759DF) - (i >> 1), jnp.float32)
    for _ in range(3):
        y = y * (1.5 - 0.5 * x * y * y)
    return y


@functools.partial(
    pl.kernel,
    out_type=(
        jax.ShapeDtypeStruct((E_PAD // C, C), jnp.float32),
        jax.ShapeDtypeStruct((N_PAD,), jnp.float32),
    ),
    mesh=_mesh,
    compiler_params=pltpu.CompilerParams(needs_layout_passes=False),
    scratch_types=(
        pltpu.VMEM((ECH_SC, C), jnp.int32),
        pltpu.VMEM((ECH_SC, C), jnp.float32),
        pltpu.VMEM((ECH_G, C), jnp.int32),
        pltpu.VMEM((ECH_G, C), jnp.float32),
        pltpu.VMEM((N_PAD,), jnp.float32),
        pltpu.VMEM((N_PAD // 16,), jnp.float32),
        pltpu.VMEM_SHARED((N_PAD,), jnp.float32),
        pltpu.VMEM_SHARED((N_PAD,), jnp.float32),
    ),
)
def _prep(row_hbm, col_hbm, ew_hbm, norm_hbm, sw_hbm,
          col_v, ew_v, row_v, norm_v, dis_v, node_v, deg_sp, dis_sp):
    cid = lax.axis_index("c")
    sid = lax.axis_index("s")
    wid = sid * NC + cid

    def _init_body(k, _):
        node_v[pl.ds(k * L, L)] = jnp.full((L,), 1.0, jnp.float32)
        return 0
    lax.fori_loop(0, ROWS_PER_TILE // L, _init_body, 0)
    pltpu.sync_copy(node_v, deg_sp.at[pl.ds(sid * ROWS_PER_TILE, ROWS_PER_TILE)])
    plsc.subcore_barrier()

    pltpu.sync_copy(col_hbm.at[pl.ds(sid * ECH_SC, ECH_SC)], col_v)
    pltpu.sync_copy(ew_hbm.at[pl.ds(sid * ECH_SC, ECH_SC)], ew_v)

    def _deg_body(j, _):
        pltpu.sync_copy(ew_v.at[j], deg_sp.at[col_v.at[j]], add=True)
        return 0
    lax.fori_loop(0, ECH_SC, _deg_body, 0)
    plsc.subcore_barrier()

    base = sid * ROWS_PER_TILE
    pltpu.sync_copy(deg_sp.at[pl.ds(base, ROWS_PER_TILE)], node_v)

    def _dis_body(k, _):
        x = node_v[pl.ds(k * L, L)]
        node_v[pl.ds(k * L, L)] = _rsqrt_newton(x)
        return 0
    lax.fori_loop(0, ROWS_PER_TILE // L, _dis_body, 0)
    pltpu.sync_copy(node_v, dis_sp.at[pl.ds(base, ROWS_PER_TILE)])

    @pl.when(cid == 0)
    def _():
        def _sw_body(k, _):
            y = node_v[pl.ds(k * L, L)]
            node_v[pl.ds(k * L, L)] = y * y
            return 0
        lax.fori_loop(0, ROWS_PER_TILE // L, _sw_body, 0)
        pltpu.sync_copy(node_v, sw_hbm.at[pl.ds(base, ROWS_PER_TILE)])
    plsc.subcore_barrier()

    pltpu.sync_copy(dis_sp, dis_v)
    ebase = wid * ECH_G
    pltpu.sync_copy(row_hbm.at[pl.ds(ebase, ECH_G)], row_v)
    pltpu.sync_copy(col_hbm.at[pl.ds(ebase, ECH_G)], col_v.at[pl.ds(0, ECH_G)])
    pltpu.sync_copy(ew_hbm.at[pl.ds(ebase, ECH_G)], ew_v.at[pl.ds(0, ECH_G)])

    def _norm_body(j, _):
        def _inner(m, _):
            r = row_v[j, pl.ds(m * L, L)]
            c = col_v[j, pl.ds(m * L, L)]
            w = ew_v[j, pl.ds(m * L, L)]
            dr = plsc.load_gather(dis_v, [r])
            dc = plsc.load_gather(dis_v, [c])
            norm_v[j, pl.ds(m * L, L)] = dr * w * dc
            return 0
        lax.fori_loop(0, C // L, _inner, 0)
        return 0
    lax.fori_loop(0, ECH_G, _norm_body, 0)
    pltpu.sync_copy(norm_v, norm_hbm.at[pl.ds(ebase, ECH_G)])


@functools.partial(
    pl.kernel,
    out_type=jax.ShapeDtypeStruct((NC, N_PAD, D), jnp.float32),
    mesh=_mesh,
    compiler_params=pltpu.CompilerParams(needs_layout_passes=False),
    scratch_types=(
        pltpu.VMEM((4, 2, C), jnp.int32),
        pltpu.VMEM((4, C), jnp.float32),
        pltpu.VMEM((2, C, D), jnp.float32),
        pltpu.VMEM_SHARED((N_PAD, D), jnp.float32),
        (pltpu.SemaphoreType.DMA,) * 4,
        (pltpu.SemaphoreType.DMA,) * 2,
        (pltpu.SemaphoreType.DMA,) * 2,
    ),
)
def _hop(h_hbm, meta_hbm, norm_hbm, out_hbm, meta_v, norm_v, rows_v, acc_sp,
         msem, gsem, ssem):
    cid = lax.axis_index("c")
    sid = lax.axis_index("s")
    wid = sid * NC + cid

    def _z_body(i, _):
        for cc in range(D // L):
            rows_v[0, i, pl.ds(cc * L, L)] = jnp.zeros((L,), jnp.float32)
        return 0
    lax.fori_loop(0, C, _z_body, 0)
    nbase = sid * ROWS_PER_TILE
    for t in range(ROWS_PER_TILE // C):
        pltpu.sync_copy(rows_v.at[0], acc_sp.at[pl.ds(nbase + t * C, C)])
    plsc.subcore_barrier()

    ebase = wid * ECH_G
    for t in range(3):
        pltpu.async_copy(meta_hbm.at[ebase + t], meta_v.at[t], msem[t])
        pltpu.async_copy(norm_hbm.at[ebase + t], norm_v.at[t], msem[t])
    pltpu.make_async_copy(meta_hbm.at[ebase], meta_v.at[0], msem[0]).wait()
    pltpu.make_async_copy(norm_hbm.at[ebase], norm_v.at[0], msem[0]).wait()
    pltpu.async_copy(h_hbm.at[meta_v.at[0, 0]], rows_v.at[0], gsem[0])

    def _quad_body(j4, _):
        for u in range(4):
            j = 4 * j4 + u
            b = u % 2
            bo = 1 - b
            m = u % 4
            mn = (u + 1) % 4
            mp = (u + 3) % 4

            pltpu.make_async_copy(h_hbm.at[meta_v.at[m, 0]], rows_v.at[b],
                                  gsem[b]).wait()

            def _scale_body(g, _):
                nv = norm_v[m, pl.ds(g * L, L)]
                for k in range(L):
                    s = nv[k]
                    r = g * L + k
                    for cc in range(D // L):
                        rows_v[b, r, pl.ds(cc * L, L)] = (
                            rows_v[b, r, pl.ds(cc * L, L)] * s)
                return 0
            lax.fori_loop(0, C // L, _scale_body, 0)

            pltpu.async_copy(rows_v.at[b], acc_sp.at[meta_v.at[m, 1]],
                             ssem[b], add=True)

            @pl.when(j >= 1)
            def _():
                pltpu.make_async_copy(rows_v.at[bo],
                                      acc_sp.at[meta_v.at[mp, 1]],
                                      ssem[bo]).wait()

            @pl.when(j + 3 < ECH_G)
            def _():
                pltpu.async_copy(meta_hbm.at[ebase + j + 3], meta_v.at[mp],
                                 msem[mp])
                pltpu.async_copy(norm_hbm.at[ebase + j + 3], norm_v.at[mp],
                                 msem[mp])

            @pl.when(j + 1 < ECH_G)
            def _():
                pltpu.make_async_copy(meta_hbm.at[ebase + j + 1],
                                      meta_v.at[mn], msem[mn]).wait()
                pltpu.make_async_copy(norm_hbm.at[ebase + j + 1],
                                      norm_v.at[mn], msem[mn]).wait()
                pltpu.async_copy(h_hbm.at[meta_v.at[mn, 0]], rows_v.at[bo],
                                 gsem[bo])
        return 0
    lax.fori_loop(0, ECH_G // 4, _quad_body, 0)
    pltpu.make_async_copy(rows_v.at[1], acc_sp.at[meta_v.at[3, 1]],
                          ssem[1]).wait()
    plsc.subcore_barrier()

    pltpu.sync_copy(acc_sp.at[pl.ds(nbase, ROWS_PER_TILE)],
                    out_hbm.at[cid].at[pl.ds(nbase, ROWS_PER_TILE)])


_ROWS_BLK = 1024


def _combine_body(p_ref, sw_ref, h_ref, o_ref):
    o_ref[...] = p_ref[0] + p_ref[1] + sw_ref[...] * h_ref[...]


def _final_body(p_ref, sw_ref, h_ref, w_ref, b_ref, o_ref):
    h2 = p_ref[0] + p_ref[1] + sw_ref[...] * h_ref[...]
    o_ref[...] = lax.dot_general(
        h2, w_ref[...], (((1,), (1,)), ((), ())),
        preferred_element_type=jnp.float32) + b_ref[...]


def _combine(p, sw, h):
    grid = (N_PAD // _ROWS_BLK,)
    return pl.pallas_call(
        _combine_body,
        grid=grid,
        in_specs=[
            pl.BlockSpec((NC, _ROWS_BLK, D), lambda i: (0, i, 0)),
            pl.BlockSpec((_ROWS_BLK, 1), lambda i: (i, 0)),
            pl.BlockSpec((_ROWS_BLK, D), lambda i: (i, 0)),
        ],
        out_specs=pl.BlockSpec((_ROWS_BLK, D), lambda i: (i, 0)),
        out_shape=jax.ShapeDtypeStruct((N_PAD, D), jnp.float32),
    )(p, sw, h)


def _final(p, sw, h, W, b):
    grid = (N_PAD // _ROWS_BLK,)
    return pl.pallas_call(
        _final_body,
        grid=grid,
        in_specs=[
            pl.BlockSpec((NC, _ROWS_BLK, D), lambda i: (0, i, 0)),
            pl.BlockSpec((_ROWS_BLK, 1), lambda i: (i, 0)),
            pl.BlockSpec((_ROWS_BLK, D), lambda i: (i, 0)),
            pl.BlockSpec((D, D), lambda i: (0, 0)),
            pl.BlockSpec((1, D), lambda i: (0, 0)),
        ],
        out_specs=pl.BlockSpec((_ROWS_BLK, D), lambda i: (i, 0)),
        out_shape=jax.ShapeDtypeStruct((N_PAD, D), jnp.float32),
    )(p, sw, h, W, b)


@jax.jit
def kernel(x, edge_index, edge_weight, W, b):
    npad = E_PAD - N_EDGES
    pad_idx = (jnp.arange(npad, dtype=jnp.int32) * 37) % N_NODES
    row = jnp.concatenate([edge_index[0].astype(jnp.int32), pad_idx])
    col = jnp.concatenate([edge_index[1].astype(jnp.int32), pad_idx])
    ew = jnp.concatenate([edge_weight, jnp.zeros((npad,), jnp.float32)])
    row2 = row.reshape(E_PAD // C, C)
    col2 = col.reshape(E_PAD // C, C)
    ew2 = ew.reshape(E_PAD // C, C)

    norm, sw = _prep(row2, col2, ew2)
    sw2 = sw.reshape(N_PAD, 1)
    meta = jnp.concatenate([row2[:, None, :], col2[:, None, :]], axis=1)

    h0 = jnp.zeros((N_PAD, D), jnp.float32).at[:N_NODES].set(x)
    p = _hop(h0, meta, norm)
    h1 = _combine(p, sw2, h0)
    q = _hop(h1, meta, norm)
    out = _final(q, sw2, h1, W, b.reshape(1, D))
    return out[:N_NODES]


# speedup vs baseline: 27.3809x; 1.2487x over previous
import functools

import jax
import jax.numpy as jnp
from jax import lax
from jax.experimental import pallas as pl
from jax.experimental.pallas import tpu as pltpu
from jax.experimental.pallas import tpu_sc as plsc

N_NODES = 10000
N_EDGES = 320000
D = 128

NC = 2
NS = 16
NW = NC * NS
L = 16

C = 80
N_PAD = 10240
E_PAD = 327680
ROWS_PER_TILE = N_PAD // NS
ECH_G = E_PAD // NW // C
ECH_SC = E_PAD // NS // C

_mesh = plsc.VectorSubcoreMesh(core_axis_name="c", subcore_axis_name="s")


def _rsqrt_newton(x):
    i = lax.bitcast_convert_type(x, jnp.int32)
    y = lax.bitcast_convert_type(jnp.int32(0x5F3759DF) - (i >> 1), jnp.float32)
    for _ in range(3):
        y = y * (1.5 - 0.5 * x * y * y)
    return y


@functools.partial(
    pl.kernel,
    out_type=(
        jax.ShapeDtypeStruct((E_PAD // C, C), jnp.float32),
        jax.ShapeDtypeStruct((N_PAD,), jnp.float32),
    ),
    mesh=_mesh,
    compiler_params=pltpu.CompilerParams(needs_layout_passes=False),
    scratch_types=(
        pltpu.VMEM((ECH_SC, C), jnp.int32),
        pltpu.VMEM((ECH_SC, C), jnp.float32),
        pltpu.VMEM((ECH_G, C), jnp.int32),
        pltpu.VMEM((ECH_G, C), jnp.float32),
        pltpu.VMEM((N_PAD,), jnp.float32),
        pltpu.VMEM((N_PAD // 16,), jnp.float32),
        pltpu.VMEM_SHARED((N_PAD,), jnp.float32),
        pltpu.VMEM_SHARED((N_PAD,), jnp.float32),
        pltpu.SemaphoreType.DMA,
    ),
)
def _prep(row_hbm, col_hbm, ew_hbm, norm_hbm, sw_hbm,
          col_v, ew_v, row_v, norm_v, dis_v, node_v, deg_sp, dis_sp, dsem):
    cid = lax.axis_index("c")
    sid = lax.axis_index("s")
    wid = sid * NC + cid

    def _init_body(k, _):
        node_v[pl.ds(k * L, L)] = jnp.full((L,), 1.0, jnp.float32)
        return 0
    lax.fori_loop(0, ROWS_PER_TILE // L, _init_body, 0)
    pltpu.sync_copy(node_v, deg_sp.at[pl.ds(sid * ROWS_PER_TILE, ROWS_PER_TILE)])
    plsc.subcore_barrier()

    pltpu.sync_copy(col_hbm.at[pl.ds(sid * ECH_SC, ECH_SC)], col_v)
    pltpu.sync_copy(ew_hbm.at[pl.ds(sid * ECH_SC, ECH_SC)], ew_v)

    def _deg_body(j, _):
        pltpu.async_copy(ew_v.at[j], deg_sp.at[col_v.at[j]], dsem, add=True)

        @pl.when(j >= 8)
        def _():
            pltpu.make_async_copy(ew_v.at[0], deg_sp.at[col_v.at[0]], dsem).wait()
        return 0
    lax.fori_loop(0, ECH_SC, _deg_body, 0)
    for _ in range(8):
        pltpu.make_async_copy(ew_v.at[0], deg_sp.at[col_v.at[0]], dsem).wait()
    plsc.subcore_barrier()

    base = sid * ROWS_PER_TILE
    pltpu.sync_copy(deg_sp.at[pl.ds(base, ROWS_PER_TILE)], node_v)

    def _dis_body(k, _):
        x = node_v[pl.ds(k * L, L)]
        node_v[pl.ds(k * L, L)] = _rsqrt_newton(x)
        return 0
    lax.fori_loop(0, ROWS_PER_TILE // L, _dis_body, 0)
    pltpu.sync_copy(node_v, dis_sp.at[pl.ds(base, ROWS_PER_TILE)])

    @pl.when(cid == 0)
    def _():
        def _sw_body(k, _):
            y = node_v[pl.ds(k * L, L)]
            node_v[pl.ds(k * L, L)] = y * y
            return 0
        lax.fori_loop(0, ROWS_PER_TILE // L, _sw_body, 0)
        pltpu.sync_copy(node_v, sw_hbm.at[pl.ds(base, ROWS_PER_TILE)])
    plsc.subcore_barrier()

    pltpu.sync_copy(dis_sp, dis_v)
    ebase = wid * ECH_G
    pltpu.sync_copy(row_hbm.at[pl.ds(ebase, ECH_G)], row_v)
    pltpu.sync_copy(col_hbm.at[pl.ds(ebase, ECH_G)], col_v.at[pl.ds(0, ECH_G)])
    pltpu.sync_copy(ew_hbm.at[pl.ds(ebase, ECH_G)], ew_v.at[pl.ds(0, ECH_G)])

    def _norm_body(j, _):
        def _inner(m, _):
            r = row_v[j, pl.ds(m * L, L)]
            c = col_v[j, pl.ds(m * L, L)]
            w = ew_v[j, pl.ds(m * L, L)]
            dr = plsc.load_gather(dis_v, [r])
            dc = plsc.load_gather(dis_v, [c])
            norm_v[j, pl.ds(m * L, L)] = dr * w * dc
            return 0
        lax.fori_loop(0, C // L, _inner, 0)
        return 0
    lax.fori_loop(0, ECH_G, _norm_body, 0)
    pltpu.sync_copy(norm_v, norm_hbm.at[pl.ds(ebase, ECH_G)])


@functools.partial(
    pl.kernel,
    out_type=jax.ShapeDtypeStruct((NC, N_PAD, D), jnp.float32),
    mesh=_mesh,
    compiler_params=pltpu.CompilerParams(needs_layout_passes=False),
    scratch_types=(
        pltpu.VMEM((8, 2, C), jnp.int32),
        pltpu.VMEM((8, C), jnp.float32),
        pltpu.VMEM((4, C, D), jnp.float32),
        pltpu.VMEM_SHARED((N_PAD, D), jnp.float32),
        (pltpu.SemaphoreType.DMA,) * 8,
        (pltpu.SemaphoreType.DMA,) * 4,
        (pltpu.SemaphoreType.DMA,) * 4,
    ),
)
def _hop(h_hbm, meta_hbm, norm_hbm, out_hbm, meta_v, norm_v, rows_v, acc_sp,
         msem, gsem, ssem):
    cid = lax.axis_index("c")
    sid = lax.axis_index("s")
    wid = sid * NC + cid

    def _z_body(i, _):
        for cc in range(D // L):
            rows_v[0, i, pl.ds(cc * L, L)] = jnp.zeros((L,), jnp.float32)
        return 0
    lax.fori_loop(0, C, _z_body, 0)
    nbase = sid * ROWS_PER_TILE
    for t in range(ROWS_PER_TILE // C):
        pltpu.sync_copy(rows_v.at[0], acc_sp.at[pl.ds(nbase + t * C, C)])
    plsc.subcore_barrier()

    ebase = wid * ECH_G

    def _meta_issue(j, slot):
        pltpu.async_copy(meta_hbm.at[ebase + j], meta_v.at[slot], msem[slot])
        pltpu.async_copy(norm_hbm.at[ebase + j], norm_v.at[slot], msem[slot])

    def _meta_wait(slot):
        pltpu.make_async_copy(meta_hbm.at[ebase], meta_v.at[slot],
                              msem[slot]).wait()
        pltpu.make_async_copy(norm_hbm.at[ebase], norm_v.at[slot],
                              msem[slot]).wait()

    for t in range(6):
        _meta_issue(t, t)
    for t in range(2):
        _meta_wait(t)
        pltpu.async_copy(h_hbm.at[meta_v.at[t, 0]], rows_v.at[t], gsem[t])

    def _oct_body(j8, _):
        for u in range(8):
            j = 8 * j8 + u
            b = u % 4
            bn = (u + 2) % 4
            mm = u % 8
            m2 = (u + 2) % 8
            m6 = (u + 6) % 8

            pltpu.make_async_copy(h_hbm.at[meta_v.at[mm, 0]], rows_v.at[b],
                                  gsem[b]).wait()

            def _scale_body(g, _):
                nv = norm_v[mm, pl.ds(g * L, L)]
                for k in range(L):
                    s = nv[k]
                    r = g * L + k
                    for cc in range(D // L):
                        rows_v[b, r, pl.ds(cc * L, L)] = (
                            rows_v[b, r, pl.ds(cc * L, L)] * s)
                return 0
            lax.fori_loop(0, C // L, _scale_body, 0)

            pltpu.async_copy(rows_v.at[b], acc_sp.at[meta_v.at[mm, 1]],
                             ssem[b], add=True)

            @pl.when(j >= 2)
            def _():
                pltpu.make_async_copy(rows_v.at[bn],
                                      acc_sp.at[meta_v.at[m6, 1]],
                                      ssem[bn]).wait()

            @pl.when(j + 2 < ECH_G)
            def _():
                _meta_wait(m2)
                pltpu.async_copy(h_hbm.at[meta_v.at[m2, 0]], rows_v.at[bn],
                                 gsem[bn])

            @pl.when(j + 6 < ECH_G)
            def _():
                _meta_issue(j + 6, m6)
        return 0
    lax.fori_loop(0, ECH_G // 8, _oct_body, 0)
    pltpu.make_async_copy(rows_v.at[2], acc_sp.at[meta_v.at[6, 1]],
                          ssem[2]).wait()
    pltpu.make_async_copy(rows_v.at[3], acc_sp.at[meta_v.at[7, 1]],
                          ssem[3]).wait()
    plsc.subcore_barrier()

    pltpu.sync_copy(acc_sp.at[pl.ds(nbase, ROWS_PER_TILE)],
                    out_hbm.at[cid].at[pl.ds(nbase, ROWS_PER_TILE)])


_ROWS_BLK = 1024


def _combine_body(p_ref, sw_ref, h_ref, o_ref):
    o_ref[...] = p_ref[0] + p_ref[1] + sw_ref[...] * h_ref[...]


def _final_body(p_ref, sw_ref, h_ref, w_ref, b_ref, o_ref):
    h2 = p_ref[0] + p_ref[1] + sw_ref[...] * h_ref[...]
    o_ref[...] = lax.dot_general(
        h2, w_ref[...], (((1,), (1,)), ((), ())),
        preferred_element_type=jnp.float32) + b_ref[...]


def _combine(p, sw, h):
    grid = (N_PAD // _ROWS_BLK,)
    return pl.pallas_call(
        _combine_body,
        grid=grid,
        in_specs=[
            pl.BlockSpec((NC, _ROWS_BLK, D), lambda i: (0, i, 0)),
            pl.BlockSpec((_ROWS_BLK, 1), lambda i: (i, 0)),
            pl.BlockSpec((_ROWS_BLK, D), lambda i: (i, 0)),
        ],
        out_specs=pl.BlockSpec((_ROWS_BLK, D), lambda i: (i, 0)),
        out_shape=jax.ShapeDtypeStruct((N_PAD, D), jnp.float32),
    )(p, sw, h)


def _final(p, sw, h, W, b):
    grid = (N_PAD // _ROWS_BLK,)
    return pl.pallas_call(
        _final_body,
        grid=grid,
        in_specs=[
            pl.BlockSpec((NC, _ROWS_BLK, D), lambda i: (0, i, 0)),
            pl.BlockSpec((_ROWS_BLK, 1), lambda i: (i, 0)),
            pl.BlockSpec((_ROWS_BLK, D), lambda i: (i, 0)),
            pl.BlockSpec((D, D), lambda i: (0, 0)),
            pl.BlockSpec((1, D), lambda i: (0, 0)),
        ],
        out_specs=pl.BlockSpec((_ROWS_BLK, D), lambda i: (i, 0)),
        out_shape=jax.ShapeDtypeStruct((N_PAD, D), jnp.float32),
    )(p, sw, h, W, b)


@jax.jit
def kernel(x, edge_index, edge_weight, W, b):
    npad = E_PAD - N_EDGES
    pad_idx = (jnp.arange(npad, dtype=jnp.int32) * 37) % N_NODES
    row = jnp.concatenate([edge_index[0].astype(jnp.int32), pad_idx])
    col = jnp.concatenate([edge_index[1].astype(jnp.int32), pad_idx])
    ew = jnp.concatenate([edge_weight, jnp.zeros((npad,), jnp.float32)])
    row2 = row.reshape(E_PAD // C, C)
    col2 = col.reshape(E_PAD // C, C)
    ew2 = ew.reshape(E_PAD // C, C)

    norm, sw = _prep(row2, col2, ew2)
    sw2 = sw.reshape(N_PAD, 1)
    meta = jnp.concatenate([row2[:, None, :], col2[:, None, :]], axis=1)

    h0 = jnp.zeros((N_PAD, D), jnp.float32).at[:N_NODES].set(x)
    p = _hop(h0, meta, norm)
    h1 = _combine(p, sw2, h0)
    q = _hop(h1, meta, norm)
    out = _final(q, sw2, h1, W, b.reshape(1, D))
    return out[:N_NODES]


# speedup vs baseline: 30.7362x; 1.1225x over previous
import functools

import jax
import jax.numpy as jnp
from jax import lax
from jax.experimental import pallas as pl
from jax.experimental.pallas import tpu as pltpu
from jax.experimental.pallas import tpu_sc as plsc

N_NODES = 10000
N_EDGES = 320000
D = 128

NC = 2
NS = 16
NW = NC * NS
L = 16

C = 80
N_PAD = 10240
E_PAD = 327680
ROWS_PER_TILE = N_PAD // NS
ECH_G = E_PAD // NW // C
ECH_SC = E_PAD // NS // C

_mesh = plsc.VectorSubcoreMesh(core_axis_name="c", subcore_axis_name="s")


def _rsqrt_newton(x):
    i = lax.bitcast_convert_type(x, jnp.int32)
    y = lax.bitcast_convert_type(jnp.int32(0x5F3759DF) - (i >> 1), jnp.float32)
    for _ in range(3):
        y = y * (1.5 - 0.5 * x * y * y)
    return y


@functools.partial(
    pl.kernel,
    out_type=(
        jax.ShapeDtypeStruct((E_PAD // C, C), jnp.float32),
        jax.ShapeDtypeStruct((N_PAD,), jnp.float32),
    ),
    mesh=_mesh,
    compiler_params=pltpu.CompilerParams(needs_layout_passes=False),
    scratch_types=(
        pltpu.VMEM((ECH_SC, C), jnp.int32),
        pltpu.VMEM((ECH_SC, C), jnp.float32),
        pltpu.VMEM((ECH_G, C), jnp.int32),
        pltpu.VMEM((ECH_G, C), jnp.float32),
        pltpu.VMEM((N_PAD,), jnp.float32),
        pltpu.VMEM((N_PAD // 16,), jnp.float32),
        pltpu.VMEM_SHARED((N_PAD,), jnp.float32),
        pltpu.VMEM_SHARED((N_PAD,), jnp.float32),
        pltpu.SemaphoreType.DMA,
    ),
)
def _prep(row_hbm, col_hbm, ew_hbm, norm_hbm, sw_hbm,
          col_v, ew_v, row_v, norm_v, dis_v, node_v, deg_sp, dis_sp, dsem):
    cid = lax.axis_index("c")
    sid = lax.axis_index("s")
    wid = sid * NC + cid

    def _init_body(k, _):
        node_v[pl.ds(k * L, L)] = jnp.full((L,), 1.0, jnp.float32)
        return 0
    lax.fori_loop(0, ROWS_PER_TILE // L, _init_body, 0)
    pltpu.sync_copy(node_v, deg_sp.at[pl.ds(sid * ROWS_PER_TILE, ROWS_PER_TILE)])
    plsc.subcore_barrier()

    pltpu.sync_copy(col_hbm.at[pl.ds(sid * ECH_SC, ECH_SC)], col_v)
    pltpu.sync_copy(ew_hbm.at[pl.ds(sid * ECH_SC, ECH_SC)], ew_v)

    def _deg_body(j, _):
        pltpu.async_copy(ew_v.at[j], deg_sp.at[col_v.at[j]], dsem, add=True)

        @pl.when(j >= 8)
        def _():
            pltpu.make_async_copy(ew_v.at[0], deg_sp.at[col_v.at[0]], dsem).wait()
        return 0
    lax.fori_loop(0, ECH_SC, _deg_body, 0)
    for _ in range(8):
        pltpu.make_async_copy(ew_v.at[0], deg_sp.at[col_v.at[0]], dsem).wait()
    plsc.subcore_barrier()

    base = sid * ROWS_PER_TILE
    pltpu.sync_copy(deg_sp.at[pl.ds(base, ROWS_PER_TILE)], node_v)

    def _dis_body(k, _):
        x = node_v[pl.ds(k * L, L)]
        node_v[pl.ds(k * L, L)] = _rsqrt_newton(x)
        return 0
    lax.fori_loop(0, ROWS_PER_TILE // L, _dis_body, 0)
    pltpu.sync_copy(node_v, dis_sp.at[pl.ds(base, ROWS_PER_TILE)])

    @pl.when(cid == 0)
    def _():
        def _sw_body(k, _):
            y = node_v[pl.ds(k * L, L)]
            node_v[pl.ds(k * L, L)] = y * y
            return 0
        lax.fori_loop(0, ROWS_PER_TILE // L, _sw_body, 0)
        pltpu.sync_copy(node_v, sw_hbm.at[pl.ds(base, ROWS_PER_TILE)])
    plsc.subcore_barrier()

    pltpu.sync_copy(dis_sp, dis_v)
    ebase = wid * ECH_G
    pltpu.sync_copy(row_hbm.at[pl.ds(ebase, ECH_G)], row_v)
    pltpu.sync_copy(col_hbm.at[pl.ds(ebase, ECH_G)], col_v.at[pl.ds(0, ECH_G)])
    pltpu.sync_copy(ew_hbm.at[pl.ds(ebase, ECH_G)], ew_v.at[pl.ds(0, ECH_G)])

    def _norm_body(j, _):
        def _inner(m, _):
            r = row_v[j, pl.ds(m * L, L)]
            c = col_v[j, pl.ds(m * L, L)]
            w = ew_v[j, pl.ds(m * L, L)]
            dr = plsc.load_gather(dis_v, [r])
            dc = plsc.load_gather(dis_v, [c])
            norm_v[j, pl.ds(m * L, L)] = dr * w * dc
            return 0
        lax.fori_loop(0, C // L, _inner, 0)
        return 0
    lax.fori_loop(0, ECH_G, _norm_body, 0)
    pltpu.sync_copy(norm_v, norm_hbm.at[pl.ds(ebase, ECH_G)])


@functools.partial(
    pl.kernel,
    out_type=jax.ShapeDtypeStruct((NC, N_PAD, D), jnp.float32),
    mesh=_mesh,
    compiler_params=pltpu.CompilerParams(needs_layout_passes=False),
    scratch_types=(
        pltpu.VMEM((8, 2, C), jnp.int32),
        pltpu.VMEM((8, C), jnp.float32),
        pltpu.VMEM((4, C, D), jnp.float32),
        pltpu.VMEM_SHARED((N_PAD, D), jnp.float32),
        (pltpu.SemaphoreType.DMA,) * 8,
        (pltpu.SemaphoreType.DMA,) * 4,
        (pltpu.SemaphoreType.DMA,) * 4,
    ),
)
def _hop(h_hbm, meta_hbm, norm_hbm, out_hbm, meta_v, norm_v, rows_v, acc_sp,
         msem, gsem, ssem):
    cid = lax.axis_index("c")
    sid = lax.axis_index("s")
    wid = sid * NC + cid

    def _z_body(i, _):
        for cc in range(D // L):
            rows_v[0, i, pl.ds(cc * L, L)] = jnp.zeros((L,), jnp.float32)
        return 0
    lax.fori_loop(0, C, _z_body, 0)
    nbase = sid * ROWS_PER_TILE
    for t in range(ROWS_PER_TILE // C):
        pltpu.sync_copy(rows_v.at[0], acc_sp.at[pl.ds(nbase + t * C, C)])
    plsc.subcore_barrier()

    ebase = wid * ECH_G

    def _meta_issue(j, slot):
        pltpu.async_copy(meta_hbm.at[ebase + j], meta_v.at[slot], msem[slot])
        pltpu.async_copy(norm_hbm.at[ebase + j], norm_v.at[slot], msem[slot])

    def _meta_wait(slot):
        pltpu.make_async_copy(meta_hbm.at[ebase], meta_v.at[slot],
                              msem[slot]).wait()
        pltpu.make_async_copy(norm_hbm.at[ebase], norm_v.at[slot],
                              msem[slot]).wait()

    for t in range(6):
        _meta_issue(t, t)
    for t in range(2):
        _meta_wait(t)
        pltpu.async_copy(h_hbm.at[meta_v.at[t, 0]], rows_v.at[t], gsem[t])

    def _oct_body(j8, _):
        for u in range(8):
            j = 8 * j8 + u
            b = u % 4
            bn = (u + 2) % 4
            mm = u % 8
            m2 = (u + 2) % 8
            m6 = (u + 6) % 8

            pltpu.make_async_copy(h_hbm.at[meta_v.at[mm, 0]], rows_v.at[b],
                                  gsem[b]).wait()

            @functools.partial(plsc.parallel_loop, 0, C // L)
            def _scale_body(g):
                nv = norm_v[mm, pl.ds(g * L, L)]
                for k in range(L):
                    s = nv[k]
                    r = g * L + k
                    for cc in range(D // L):
                        rows_v[b, r, pl.ds(cc * L, L)] = (
                            rows_v[b, r, pl.ds(cc * L, L)] * s)

            pltpu.async_copy(rows_v.at[b], acc_sp.at[meta_v.at[mm, 1]],
                             ssem[b], add=True)

            @pl.when(j >= 2)
            def _():
                pltpu.make_async_copy(rows_v.at[bn],
                                      acc_sp.at[meta_v.at[m6, 1]],
                                      ssem[bn]).wait()

            @pl.when(j + 2 < ECH_G)
            def _():
                _meta_wait(m2)
                pltpu.async_copy(h_hbm.at[meta_v.at[m2, 0]], rows_v.at[bn],
                                 gsem[bn])

            @pl.when(j + 6 < ECH_G)
            def _():
                _meta_issue(j + 6, m6)
        return 0
    lax.fori_loop(0, ECH_G // 8, _oct_body, 0)
    pltpu.make_async_copy(rows_v.at[2], acc_sp.at[meta_v.at[6, 1]],
                          ssem[2]).wait()
    pltpu.make_async_copy(rows_v.at[3], acc_sp.at[meta_v.at[7, 1]],
                          ssem[3]).wait()
    plsc.subcore_barrier()

    pltpu.sync_copy(acc_sp.at[pl.ds(nbase, ROWS_PER_TILE)],
                    out_hbm.at[cid].at[pl.ds(nbase, ROWS_PER_TILE)])


_ROWS_BLK = 1024


def _combine_body(p_ref, sw_ref, h_ref, o_ref):
    o_ref[...] = p_ref[0] + p_ref[1] + sw_ref[...] * h_ref[...]


def _final_body(p_ref, sw_ref, h_ref, w_ref, b_ref, o_ref):
    h2 = p_ref[0] + p_ref[1] + sw_ref[...] * h_ref[...]
    o_ref[...] = lax.dot_general(
        h2, w_ref[...], (((1,), (1,)), ((), ())),
        preferred_element_type=jnp.float32) + b_ref[...]


def _combine(p, sw, h):
    grid = (N_PAD // _ROWS_BLK,)
    return pl.pallas_call(
        _combine_body,
        grid=grid,
        in_specs=[
            pl.BlockSpec((NC, _ROWS_BLK, D), lambda i: (0, i, 0)),
            pl.BlockSpec((_ROWS_BLK, 1), lambda i: (i, 0)),
            pl.BlockSpec((_ROWS_BLK, D), lambda i: (i, 0)),
        ],
        out_specs=pl.BlockSpec((_ROWS_BLK, D), lambda i: (i, 0)),
        out_shape=jax.ShapeDtypeStruct((N_PAD, D), jnp.float32),
    )(p, sw, h)


def _final(p, sw, h, W, b):
    grid = (N_PAD // _ROWS_BLK,)
    return pl.pallas_call(
        _final_body,
        grid=grid,
        in_specs=[
            pl.BlockSpec((NC, _ROWS_BLK, D), lambda i: (0, i, 0)),
            pl.BlockSpec((_ROWS_BLK, 1), lambda i: (i, 0)),
            pl.BlockSpec((_ROWS_BLK, D), lambda i: (i, 0)),
            pl.BlockSpec((D, D), lambda i: (0, 0)),
            pl.BlockSpec((1, D), lambda i: (0, 0)),
        ],
        out_specs=pl.BlockSpec((_ROWS_BLK, D), lambda i: (i, 0)),
        out_shape=jax.ShapeDtypeStruct((N_PAD, D), jnp.float32),
    )(p, sw, h, W, b)


@jax.jit
def kernel(x, edge_index, edge_weight, W, b):
    npad = E_PAD - N_EDGES
    pad_idx = (jnp.arange(npad, dtype=jnp.int32) * 37) % N_NODES
    row = jnp.concatenate([edge_index[0].astype(jnp.int32), pad_idx])
    col = jnp.concatenate([edge_index[1].astype(jnp.int32), pad_idx])
    ew = jnp.concatenate([edge_weight, jnp.zeros((npad,), jnp.float32)])
    row2 = row.reshape(E_PAD // C, C)
    col2 = col.reshape(E_PAD // C, C)
    ew2 = ew.reshape(E_PAD // C, C)

    norm, sw = _prep(row2, col2, ew2)
    sw2 = sw.reshape(N_PAD, 1)
    meta = jnp.concatenate([row2[:, None, :], col2[:, None, :]], axis=1)

    h0 = jnp.zeros((N_PAD, D), jnp.float32).at[:N_NODES].set(x)
    p = _hop(h0, meta, norm)
    h1 = _combine(p, sw2, h0)
    q = _hop(h1, meta, norm)
    out = _final(q, sw2, h1, W, b.reshape(1, D))
    return out[:N_NODES]


# speedup vs baseline: 31.2836x; 1.0178x over previous
import functools

import jax
import jax.numpy as jnp
from jax import lax
from jax.experimental import pallas as pl
from jax.experimental.pallas import tpu as pltpu
from jax.experimental.pallas import tpu_sc as plsc

N_NODES = 10000
N_EDGES = 320000
D = 128

NC = 2
NS = 16
NW = NC * NS
L = 16

C = 80
N_PAD = 10240
E_PAD = 327680
ROWS_PER_TILE = N_PAD // NS
ECH_G = E_PAD // NW // C
ECH_SC = E_PAD // NS // C

_mesh = plsc.VectorSubcoreMesh(core_axis_name="c", subcore_axis_name="s")


def _rsqrt_newton(x):
    i = lax.bitcast_convert_type(x, jnp.int32)
    y = lax.bitcast_convert_type(jnp.int32(0x5F3759DF) - (i >> 1), jnp.float32)
    for _ in range(3):
        y = y * (1.5 - 0.5 * x * y * y)
    return y


@functools.partial(
    pl.kernel,
    out_type=(
        jax.ShapeDtypeStruct((E_PAD // C, C), jnp.float32),
        jax.ShapeDtypeStruct((N_PAD,), jnp.float32),
    ),
    mesh=_mesh,
    compiler_params=pltpu.CompilerParams(needs_layout_passes=False),
    scratch_types=(
        pltpu.VMEM((ECH_SC, C), jnp.int32),
        pltpu.VMEM((ECH_SC, C), jnp.float32),
        pltpu.VMEM((ECH_G, C), jnp.int32),
        pltpu.VMEM((ECH_G, C), jnp.float32),
        pltpu.VMEM((N_PAD,), jnp.float32),
        pltpu.VMEM((N_PAD // 16,), jnp.float32),
        pltpu.VMEM_SHARED((N_PAD,), jnp.float32),
        pltpu.VMEM_SHARED((N_PAD,), jnp.float32),
        pltpu.SemaphoreType.DMA,
    ),
)
def _prep(row_hbm, col_hbm, ew_hbm, norm_hbm, sw_hbm,
          col_v, ew_v, row_v, norm_v, dis_v, node_v, deg_sp, dis_sp, dsem):
    cid = lax.axis_index("c")
    sid = lax.axis_index("s")
    wid = sid * NC + cid

    def _init_body(k, _):
        node_v[pl.ds(k * L, L)] = jnp.full((L,), 1.0, jnp.float32)
        return 0
    lax.fori_loop(0, ROWS_PER_TILE // L, _init_body, 0)
    pltpu.sync_copy(node_v, deg_sp.at[pl.ds(sid * ROWS_PER_TILE, ROWS_PER_TILE)])
    plsc.subcore_barrier()

    pltpu.sync_copy(col_hbm.at[pl.ds(sid * ECH_SC, ECH_SC)], col_v)
    pltpu.sync_copy(ew_hbm.at[pl.ds(sid * ECH_SC, ECH_SC)], ew_v)

    def _deg_body(j, _):
        pltpu.async_copy(ew_v.at[j], deg_sp.at[col_v.at[j]], dsem, add=True)

        @pl.when(j >= 8)
        def _():
            pltpu.make_async_copy(ew_v.at[0], deg_sp.at[col_v.at[0]], dsem).wait()
        return 0
    lax.fori_loop(0, ECH_SC, _deg_body, 0)
    for _ in range(8):
        pltpu.make_async_copy(ew_v.at[0], deg_sp.at[col_v.at[0]], dsem).wait()
    plsc.subcore_barrier()

    base = sid * ROWS_PER_TILE
    pltpu.sync_copy(deg_sp.at[pl.ds(base, ROWS_PER_TILE)], node_v)

    def _dis_body(k, _):
        x = node_v[pl.ds(k * L, L)]
        node_v[pl.ds(k * L, L)] = _rsqrt_newton(x)
        return 0
    lax.fori_loop(0, ROWS_PER_TILE // L, _dis_body, 0)
    pltpu.sync_copy(node_v, dis_sp.at[pl.ds(base, ROWS_PER_TILE)])

    @pl.when(cid == 0)
    def _():
        def _sw_body(k, _):
            y = node_v[pl.ds(k * L, L)]
            node_v[pl.ds(k * L, L)] = y * y
            return 0
        lax.fori_loop(0, ROWS_PER_TILE // L, _sw_body, 0)
        pltpu.sync_copy(node_v, sw_hbm.at[pl.ds(base, ROWS_PER_TILE)])
    plsc.subcore_barrier()

    pltpu.sync_copy(dis_sp, dis_v)
    ebase = wid * ECH_G
    pltpu.sync_copy(row_hbm.at[pl.ds(ebase, ECH_G)], row_v)
    pltpu.sync_copy(col_hbm.at[pl.ds(ebase, ECH_G)], col_v.at[pl.ds(0, ECH_G)])
    pltpu.sync_copy(ew_hbm.at[pl.ds(ebase, ECH_G)], ew_v.at[pl.ds(0, ECH_G)])

    @functools.partial(plsc.parallel_loop, 0, ECH_G * (C // L))
    def _norm_body(t):
        j = t // (C // L)
        m = t % (C // L)
        r = row_v[j, pl.ds(m * L, L)]
        c = col_v[j, pl.ds(m * L, L)]
        w = ew_v[j, pl.ds(m * L, L)]
        dr = plsc.load_gather(dis_v, [r])
        dc = plsc.load_gather(dis_v, [c])
        norm_v[j, pl.ds(m * L, L)] = dr * w * dc
    pltpu.sync_copy(norm_v, norm_hbm.at[pl.ds(ebase, ECH_G)])


@functools.partial(
    pl.kernel,
    out_type=jax.ShapeDtypeStruct((NC, N_PAD, D), jnp.float32),
    mesh=_mesh,
    compiler_params=pltpu.CompilerParams(needs_layout_passes=False),
    scratch_types=(
        pltpu.VMEM((8, 2, C), jnp.int32),
        pltpu.VMEM((8, C), jnp.float32),
        pltpu.VMEM((4, C, D), jnp.float32),
        pltpu.VMEM_SHARED((N_PAD, D), jnp.float32),
        (pltpu.SemaphoreType.DMA,) * 8,
        (pltpu.SemaphoreType.DMA,) * 4,
        (pltpu.SemaphoreType.DMA,) * 4,
    ),
)
def _hop(h_hbm, meta_hbm, norm_hbm, out_hbm, meta_v, norm_v, rows_v, acc_sp,
         msem, gsem, ssem):
    cid = lax.axis_index("c")
    sid = lax.axis_index("s")
    wid = sid * NC + cid

    @functools.partial(plsc.parallel_loop, 0, C)
    def _z_body(i):
        for cc in range(D // L):
            rows_v[0, i, pl.ds(cc * L, L)] = jnp.zeros((L,), jnp.float32)
    nbase = sid * ROWS_PER_TILE
    for t in range(ROWS_PER_TILE // C):
        pltpu.sync_copy(rows_v.at[0], acc_sp.at[pl.ds(nbase + t * C, C)])
    plsc.subcore_barrier()

    ebase = wid * ECH_G

    def _meta_issue(j, slot):
        pltpu.async_copy(meta_hbm.at[ebase + j], meta_v.at[slot], msem[slot])
        pltpu.async_copy(norm_hbm.at[ebase + j], norm_v.at[slot], msem[slot])

    def _meta_wait(slot):
        pltpu.make_async_copy(meta_hbm.at[ebase], meta_v.at[slot],
                              msem[slot]).wait()
        pltpu.make_async_copy(norm_hbm.at[ebase], norm_v.at[slot],
                              msem[slot]).wait()

    for t in range(6):
        _meta_issue(t, t)
    for t in range(2):
        _meta_wait(t)
        pltpu.async_copy(h_hbm.at[meta_v.at[t, 0]], rows_v.at[t], gsem[t])

    def _oct_body(j8, _):
        for u in range(8):
            j = 8 * j8 + u
            b = u % 4
            bn = (u + 2) % 4
            mm = u % 8
            m2 = (u + 2) % 8
            m6 = (u + 6) % 8

            pltpu.make_async_copy(h_hbm.at[meta_v.at[mm, 0]], rows_v.at[b],
                                  gsem[b]).wait()

            @functools.partial(plsc.parallel_loop, 0, C // L)
            def _scale_body(g):
                nv = norm_v[mm, pl.ds(g * L, L)]
                for k in range(L):
                    s = nv[k]
                    r = g * L + k
                    for cc in range(D // L):
                        rows_v[b, r, pl.ds(cc * L, L)] = (
                            rows_v[b, r, pl.ds(cc * L, L)] * s)

            pltpu.async_copy(rows_v.at[b], acc_sp.at[meta_v.at[mm, 1]],
                             ssem[b], add=True)

            @pl.when(j >= 2)
            def _():
                pltpu.make_async_copy(rows_v.at[bn],
                                      acc_sp.at[meta_v.at[m6, 1]],
                                      ssem[bn]).wait()

            @pl.when(j + 2 < ECH_G)
            def _():
                _meta_wait(m2)
                pltpu.async_copy(h_hbm.at[meta_v.at[m2, 0]], rows_v.at[bn],
                                 gsem[bn])

            @pl.when(j + 6 < ECH_G)
            def _():
                _meta_issue(j + 6, m6)
        return 0
    lax.fori_loop(0, ECH_G // 8, _oct_body, 0)
    pltpu.make_async_copy(rows_v.at[2], acc_sp.at[meta_v.at[6, 1]],
                          ssem[2]).wait()
    pltpu.make_async_copy(rows_v.at[3], acc_sp.at[meta_v.at[7, 1]],
                          ssem[3]).wait()
    plsc.subcore_barrier()

    pltpu.sync_copy(acc_sp.at[pl.ds(nbase, ROWS_PER_TILE)],
                    out_hbm.at[cid].at[pl.ds(nbase, ROWS_PER_TILE)])


_ROWS_BLK = 1024


def _combine_body(p_ref, sw_ref, h_ref, o_ref):
    o_ref[...] = p_ref[0] + p_ref[1] + sw_ref[...] * h_ref[...]


def _final_body(p_ref, sw_ref, h_ref, w_ref, b_ref, o_ref):
    h2 = p_ref[0] + p_ref[1] + sw_ref[...] * h_ref[...]
    o_ref[...] = lax.dot_general(
        h2, w_ref[...], (((1,), (1,)), ((), ())),
        preferred_element_type=jnp.float32) + b_ref[...]


def _combine(p, sw, h):
    grid = (N_PAD // _ROWS_BLK,)
    return pl.pallas_call(
        _combine_body,
        grid=grid,
        in_specs=[
            pl.BlockSpec((NC, _ROWS_BLK, D), lambda i: (0, i, 0)),
            pl.BlockSpec((_ROWS_BLK, 1), lambda i: (i, 0)),
            pl.BlockSpec((_ROWS_BLK, D), lambda i: (i, 0)),
        ],
        out_specs=pl.BlockSpec((_ROWS_BLK, D), lambda i: (i, 0)),
        out_shape=jax.ShapeDtypeStruct((N_PAD, D), jnp.float32),
    )(p, sw, h)


def _final(p, sw, h, W, b):
    grid = (N_PAD // _ROWS_BLK,)
    return pl.pallas_call(
        _final_body,
        grid=grid,
        in_specs=[
            pl.BlockSpec((NC, _ROWS_BLK, D), lambda i: (0, i, 0)),
            pl.BlockSpec((_ROWS_BLK, 1), lambda i: (i, 0)),
            pl.BlockSpec((_ROWS_BLK, D), lambda i: (i, 0)),
            pl.BlockSpec((D, D), lambda i: (0, 0)),
            pl.BlockSpec((1, D), lambda i: (0, 0)),
        ],
        out_specs=pl.BlockSpec((_ROWS_BLK, D), lambda i: (i, 0)),
        out_shape=jax.ShapeDtypeStruct((N_PAD, D), jnp.float32),
    )(p, sw, h, W, b)


@jax.jit
def kernel(x, edge_index, edge_weight, W, b):
    npad = E_PAD - N_EDGES
    pad_idx = (jnp.arange(npad, dtype=jnp.int32) * 37) % N_NODES
    row = jnp.concatenate([edge_index[0].astype(jnp.int32), pad_idx])
    col = jnp.concatenate([edge_index[1].astype(jnp.int32), pad_idx])
    ew = jnp.concatenate([edge_weight, jnp.zeros((npad,), jnp.float32)])
    row2 = row.reshape(E_PAD // C, C)
    col2 = col.reshape(E_PAD // C, C)
    ew2 = ew.reshape(E_PAD // C, C)

    norm, sw = _prep(row2, col2, ew2)
    sw2 = sw.reshape(N_PAD, 1)
    meta = jnp.concatenate([row2[:, None, :], col2[:, None, :]], axis=1)

    h0 = jnp.zeros((N_PAD, D), jnp.float32).at[:N_NODES].set(x)
    p = _hop(h0, meta, norm)
    h1 = _combine(p, sw2, h0)
    q = _hop(h1, meta, norm)
    out = _final(q, sw2, h1, W, b.reshape(1, D))
    return out[:N_NODES]


# speedup vs baseline: 31.3060x; 1.0007x over previous
import functools

import jax
import jax.numpy as jnp
from jax import lax
from jax.experimental import pallas as pl
from jax.experimental.pallas import tpu as pltpu
from jax.experimental.pallas import tpu_sc as plsc

N_NODES = 10000
N_EDGES = 320000
D = 128

NC = 2
NS = 16
NW = NC * NS
L = 16

C = 80
N_PAD = 10240
E_PAD = 327680
ROWS_PER_TILE = N_PAD // NS
ECH_G = E_PAD // NW // C
ECH_SC = E_PAD // NS // C

_mesh = plsc.VectorSubcoreMesh(core_axis_name="c", subcore_axis_name="s")


def _rsqrt_newton(x):
    i = lax.bitcast_convert_type(x, jnp.int32)
    y = lax.bitcast_convert_type(jnp.int32(0x5F3759DF) - (i >> 1), jnp.float32)
    for _ in range(3):
        y = y * (1.5 - 0.5 * x * y * y)
    return y


@functools.partial(
    pl.kernel,
    out_type=(
        jax.ShapeDtypeStruct((E_PAD // C, C), jnp.float32),
        jax.ShapeDtypeStruct((N_PAD,), jnp.float32),
    ),
    mesh=_mesh,
    compiler_params=pltpu.CompilerParams(needs_layout_passes=False),
    scratch_types=(
        pltpu.VMEM((ECH_SC, C), jnp.int32),
        pltpu.VMEM((ECH_SC, C), jnp.float32),
        pltpu.VMEM((ECH_G, C), jnp.int32),
        pltpu.VMEM((ECH_G, C), jnp.float32),
        pltpu.VMEM((N_PAD,), jnp.float32),
        pltpu.VMEM((N_PAD // 16,), jnp.float32),
        pltpu.VMEM_SHARED((N_PAD,), jnp.float32),
        pltpu.VMEM_SHARED((N_PAD,), jnp.float32),
        pltpu.SemaphoreType.DMA,
    ),
)
def _prep(row_hbm, col_hbm, ew_hbm, norm_hbm, sw_hbm,
          col_v, ew_v, row_v, norm_v, dis_v, node_v, deg_sp, dis_sp, dsem):
    cid = lax.axis_index("c")
    sid = lax.axis_index("s")
    wid = sid * NC + cid

    def _init_body(k, _):
        node_v[pl.ds(k * L, L)] = jnp.full((L,), 1.0, jnp.float32)
        return 0
    lax.fori_loop(0, ROWS_PER_TILE // L, _init_body, 0)
    pltpu.sync_copy(node_v, deg_sp.at[pl.ds(sid * ROWS_PER_TILE, ROWS_PER_TILE)])
    plsc.subcore_barrier()

    pltpu.sync_copy(col_hbm.at[pl.ds(sid * ECH_SC, ECH_SC)], col_v)
    pltpu.sync_copy(ew_hbm.at[pl.ds(sid * ECH_SC, ECH_SC)], ew_v)

    def _deg_body(j, _):
        pltpu.async_copy(ew_v.at[j], deg_sp.at[col_v.at[j]], dsem, add=True)

        @pl.when(j >= 24)
        def _():
            pltpu.make_async_copy(ew_v.at[0], deg_sp.at[col_v.at[0]], dsem).wait()
        return 0
    lax.fori_loop(0, ECH_SC, _deg_body, 0)
    for _ in range(24):
        pltpu.make_async_copy(ew_v.at[0], deg_sp.at[col_v.at[0]], dsem).wait()
    plsc.subcore_barrier()

    base = sid * ROWS_PER_TILE
    pltpu.sync_copy(deg_sp.at[pl.ds(base, ROWS_PER_TILE)], node_v)

    def _dis_body(k, _):
        x = node_v[pl.ds(k * L, L)]
        node_v[pl.ds(k * L, L)] = _rsqrt_newton(x)
        return 0
    lax.fori_loop(0, ROWS_PER_TILE // L, _dis_body, 0)
    pltpu.sync_copy(node_v, dis_sp.at[pl.ds(base, ROWS_PER_TILE)])

    @pl.when(cid == 0)
    def _():
        def _sw_body(k, _):
            y = node_v[pl.ds(k * L, L)]
            node_v[pl.ds(k * L, L)] = y * y
            return 0
        lax.fori_loop(0, ROWS_PER_TILE // L, _sw_body, 0)
        pltpu.sync_copy(node_v, sw_hbm.at[pl.ds(base, ROWS_PER_TILE)])
    plsc.subcore_barrier()

    pltpu.sync_copy(dis_sp, dis_v)
    ebase = wid * ECH_G
    pltpu.sync_copy(row_hbm.at[pl.ds(ebase, ECH_G)], row_v)
    pltpu.sync_copy(col_hbm.at[pl.ds(ebase, ECH_G)], col_v.at[pl.ds(0, ECH_G)])
    pltpu.sync_copy(ew_hbm.at[pl.ds(ebase, ECH_G)], ew_v.at[pl.ds(0, ECH_G)])

    @functools.partial(plsc.parallel_loop, 0, ECH_G * (C // L))
    def _norm_body(t):
        j = t // (C // L)
        m = t % (C // L)
        r = row_v[j, pl.ds(m * L, L)]
        c = col_v[j, pl.ds(m * L, L)]
        w = ew_v[j, pl.ds(m * L, L)]
        dr = plsc.load_gather(dis_v, [r])
        dc = plsc.load_gather(dis_v, [c])
        norm_v[j, pl.ds(m * L, L)] = dr * w * dc
    pltpu.sync_copy(norm_v, norm_hbm.at[pl.ds(ebase, ECH_G)])


@functools.partial(
    pl.kernel,
    out_type=jax.ShapeDtypeStruct((NC, N_PAD, D), jnp.float32),
    mesh=_mesh,
    compiler_params=pltpu.CompilerParams(needs_layout_passes=False),
    scratch_types=(
        pltpu.VMEM((8, 2, C), jnp.int32),
        pltpu.VMEM((8, C), jnp.float32),
        pltpu.VMEM((4, C, D), jnp.float32),
        pltpu.VMEM_SHARED((N_PAD, D), jnp.float32),
        (pltpu.SemaphoreType.DMA,) * 8,
        (pltpu.SemaphoreType.DMA,) * 4,
        (pltpu.SemaphoreType.DMA,) * 4,
    ),
)
def _hop(h_hbm, meta_hbm, norm_hbm, out_hbm, meta_v, norm_v, rows_v, acc_sp,
         msem, gsem, ssem):
    cid = lax.axis_index("c")
    sid = lax.axis_index("s")
    wid = sid * NC + cid

    @functools.partial(plsc.parallel_loop, 0, C)
    def _z_body(i):
        for cc in range(D // L):
            rows_v[0, i, pl.ds(cc * L, L)] = jnp.zeros((L,), jnp.float32)
    nbase = sid * ROWS_PER_TILE
    for t in range(ROWS_PER_TILE // C):
        pltpu.sync_copy(rows_v.at[0], acc_sp.at[pl.ds(nbase + t * C, C)])
    plsc.subcore_barrier()

    ebase = wid * ECH_G

    def _meta_issue(j, slot):
        pltpu.async_copy(meta_hbm.at[ebase + j], meta_v.at[slot], msem[slot])
        pltpu.async_copy(norm_hbm.at[ebase + j], norm_v.at[slot], msem[slot])

    def _meta_wait(slot):
        pltpu.make_async_copy(meta_hbm.at[ebase], meta_v.at[slot],
                              msem[slot]).wait()
        pltpu.make_async_copy(norm_hbm.at[ebase], norm_v.at[slot],
                              msem[slot]).wait()

    for t in range(6):
        _meta_issue(t, t)
    for t in range(2):
        _meta_wait(t)
        pltpu.async_copy(h_hbm.at[meta_v.at[t, 0]], rows_v.at[t], gsem[t])

    def _oct_body(j8, _):
        for u in range(8):
            j = 8 * j8 + u
            b = u % 4
            bn = (u + 2) % 4
            mm = u % 8
            m2 = (u + 2) % 8
            m6 = (u + 6) % 8

            pltpu.make_async_copy(h_hbm.at[meta_v.at[mm, 0]], rows_v.at[b],
                                  gsem[b]).wait()

            @functools.partial(plsc.parallel_loop, 0, C // L)
            def _scale_body(g):
                nv = norm_v[mm, pl.ds(g * L, L)]
                for k in range(L):
                    s = nv[k]
                    r = g * L + k
                    for cc in range(D // L):
                        rows_v[b, r, pl.ds(cc * L, L)] = (
                            rows_v[b, r, pl.ds(cc * L, L)] * s)

            pltpu.async_copy(rows_v.at[b], acc_sp.at[meta_v.at[mm, 1]],
                             ssem[b], add=True)

            @pl.when(j >= 2)
            def _():
                pltpu.make_async_copy(rows_v.at[bn],
                                      acc_sp.at[meta_v.at[m6, 1]],
                                      ssem[bn]).wait()

            @pl.when(j + 2 < ECH_G)
            def _():
                _meta_wait(m2)
                pltpu.async_copy(h_hbm.at[meta_v.at[m2, 0]], rows_v.at[bn],
                                 gsem[bn])

            @pl.when(j + 6 < ECH_G)
            def _():
                _meta_issue(j + 6, m6)
        return 0
    lax.fori_loop(0, ECH_G // 8, _oct_body, 0)
    pltpu.make_async_copy(rows_v.at[2], acc_sp.at[meta_v.at[6, 1]],
                          ssem[2]).wait()
    pltpu.make_async_copy(rows_v.at[3], acc_sp.at[meta_v.at[7, 1]],
                          ssem[3]).wait()
    plsc.subcore_barrier()

    pltpu.sync_copy(acc_sp.at[pl.ds(nbase, ROWS_PER_TILE)],
                    out_hbm.at[cid].at[pl.ds(nbase, ROWS_PER_TILE)])


_ROWS_BLK = 1024


def _combine_body(p_ref, sw_ref, h_ref, o_ref):
    o_ref[...] = p_ref[0] + p_ref[1] + sw_ref[...] * h_ref[...]


def _final_body(p_ref, sw_ref, h_ref, w_ref, b_ref, o_ref):
    h2 = p_ref[0] + p_ref[1] + sw_ref[...] * h_ref[...]
    o_ref[...] = lax.dot_general(
        h2, w_ref[...], (((1,), (1,)), ((), ())),
        preferred_element_type=jnp.float32) + b_ref[...]


def _combine(p, sw, h):
    grid = (N_PAD // _ROWS_BLK,)
    return pl.pallas_call(
        _combine_body,
        grid=grid,
        in_specs=[
            pl.BlockSpec((NC, _ROWS_BLK, D), lambda i: (0, i, 0)),
            pl.BlockSpec((_ROWS_BLK, 1), lambda i: (i, 0)),
            pl.BlockSpec((_ROWS_BLK, D), lambda i: (i, 0)),
        ],
        out_specs=pl.BlockSpec((_ROWS_BLK, D), lambda i: (i, 0)),
        out_shape=jax.ShapeDtypeStruct((N_PAD, D), jnp.float32),
    )(p, sw, h)


def _final(p, sw, h, W, b):
    grid = (N_PAD // _ROWS_BLK,)
    return pl.pallas_call(
        _final_body,
        grid=grid,
        in_specs=[
            pl.BlockSpec((NC, _ROWS_BLK, D), lambda i: (0, i, 0)),
            pl.BlockSpec((_ROWS_BLK, 1), lambda i: (i, 0)),
            pl.BlockSpec((_ROWS_BLK, D), lambda i: (i, 0)),
            pl.BlockSpec((D, D), lambda i: (0, 0)),
            pl.BlockSpec((1, D), lambda i: (0, 0)),
        ],
        out_specs=pl.BlockSpec((_ROWS_BLK, D), lambda i: (i, 0)),
        out_shape=jax.ShapeDtypeStruct((N_PAD, D), jnp.float32),
    )(p, sw, h, W, b)


@jax.jit
def kernel(x, edge_index, edge_weight, W, b):
    npad = E_PAD - N_EDGES
    pad_idx = (jnp.arange(npad, dtype=jnp.int32) * 37) % N_NODES
    row = jnp.concatenate([edge_index[0].astype(jnp.int32), pad_idx])
    col = jnp.concatenate([edge_index[1].astype(jnp.int32), pad_idx])
    ew = jnp.concatenate([edge_weight, jnp.zeros((npad,), jnp.float32)])
    row2 = row.reshape(E_PAD // C, C)
    col2 = col.reshape(E_PAD // C, C)
    ew2 = ew.reshape(E_PAD // C, C)

    norm, sw = _prep(row2, col2, ew2)
    sw2 = sw.reshape(N_PAD, 1)
    meta = jnp.concatenate([row2[:, None, :], col2[:, None, :]], axis=1)

    h0 = jnp.zeros((N_PAD, D), jnp.float32).at[:N_NODES].set(x)
    p = _hop(h0, meta, norm)
    h1 = _combine(p, sw2, h0)
    q = _hop(h1, meta, norm)
    out = _final(q, sw2, h1, W, b.reshape(1, D))
    return out[:N_NODES]
